# Initial kernel scaffold; baseline (speedup 1.0000x reference)
import jax
import jax.numpy as jnp
from jax.experimental import pallas as pl


def kernel(sku_id, cat_id, price_id, word_ids, sku_table, sku_ln_g, sku_ln_b, proj_W, proj_b, proj_ln_g, proj_ln_b, cat_table, cat_ln_g, cat_ln_b, price_table, price_ln_g, price_ln_b, word_table, fc1_W, fc1_b):
    raise NotImplementedError("write your pallas kernel here")


# speedup vs baseline: 2.3086x; 2.3086x over previous
import functools

import jax
import jax.numpy as jnp
from jax import lax
from jax.experimental import pallas as pl
from jax.experimental.pallas import tpu as pltpu
from jax.experimental.pallas import tpu_sc as plsc

B, L = 4096, 50
N = B * L
SKU_DIM, HID, ITEM_DIM = 64, 128, 128

NW = 32
CHUNK = 128
PER_W = N // NW
NCHUNK = PER_W // CHUNK

_EPS = 1e-5


def _ln(x, g, b):
    mu = jnp.mean(x, axis=-1, keepdims=True)
    var = jnp.mean((x - mu) ** 2, axis=-1, keepdims=True)
    return (x - mu) * lax.rsqrt(var + _EPS) * g + b


_WBLK = 2000


def _prep_body(cat_t, cat_g, cat_b, price_t, price_g, price_b,
               word_t, fc1_w, fc1_b, c2, p2, w2):
    w2[...] = jnp.dot(word_t[...], fc1_w[384:512, :],
                      preferred_element_type=jnp.float32)

    @pl.when(pl.program_id(0) == 0)
    def _():
        c2[...] = jnp.dot(_ln(cat_t[...], cat_g[...], cat_b[...]),
                          fc1_w[128:256, :],
                          preferred_element_type=jnp.float32) + fc1_b[...]
        p2[...] = jnp.dot(_ln(price_t[...], price_g[...], price_b[...]),
                          fc1_w[256:384, :],
                          preferred_element_type=jnp.float32)


def _prep(cat_t, cat_g, cat_b, price_t, price_g, price_b, word_t, fc1_w, fc1_b):
    n_cat, n_price, n_word = cat_t.shape[0], price_t.shape[0], word_t.shape[0]
    grid = n_word // _WBLK
    full = lambda shape: pl.BlockSpec(shape, lambda i: (0, 0))
    return pl.pallas_call(
        _prep_body,
        grid=(grid,),
        in_specs=[
            full((n_cat, HID)), full((1, HID)), full((1, HID)),
            full((n_price, HID)), full((1, HID)), full((1, HID)),
            pl.BlockSpec((_WBLK, HID), lambda i: (i, 0)),
            full((3 * HID + ITEM_DIM, ITEM_DIM)), full((1, ITEM_DIM)),
        ],
        out_specs=[
            full((n_cat, ITEM_DIM)), full((n_price, ITEM_DIM)),
            pl.BlockSpec((_WBLK, ITEM_DIM), lambda i: (i, 0)),
        ],
        out_shape=[
            jax.ShapeDtypeStruct((n_cat, ITEM_DIM), jnp.float32),
            jax.ShapeDtypeStruct((n_price, ITEM_DIM), jnp.float32),
            jax.ShapeDtypeStruct((n_word, ITEM_DIM), jnp.float32),
        ],
    )(cat_t, cat_g.reshape(1, HID), cat_b.reshape(1, HID),
      price_t, price_g.reshape(1, HID), price_b.reshape(1, HID),
      word_t, fc1_w, fc1_b.reshape(1, ITEM_DIM))


def _gather_body(sku_idx, cat_idx, price_idx, word_idx,
                 sku_t, c2, p2, w2,
                 sku_out, cat_out, price_out, word_out,
                 idx_s, idx_c, idx_p, idx_w, sbuf, buf, sem):
    wid = lax.axis_index("s") * 2 + lax.axis_index("c")
    base = wid * PER_W
    pltpu.sync_copy(sku_idx.at[wid], idx_s)
    pltpu.sync_copy(cat_idx.at[wid], idx_c)
    pltpu.sync_copy(price_idx.at[wid], idx_p)
    pltpu.sync_copy(word_idx.at[wid], idx_w)

    def chunk(c, carry):
        off = base + c * CHUNK
        pltpu.async_copy(sku_t.at[idx_s.at[c]], sbuf, sem).wait()
        pltpu.sync_copy(sbuf, sku_out.at[pl.ds(off, CHUNK)])
        pltpu.async_copy(c2.at[idx_c.at[c]], buf, sem).wait()
        pltpu.sync_copy(buf, cat_out.at[pl.ds(off, CHUNK)])
        pltpu.async_copy(p2.at[idx_p.at[c]], buf, sem).wait()
        pltpu.sync_copy(buf, price_out.at[pl.ds(off, CHUNK)])
        pltpu.async_copy(w2.at[idx_w.at[c]], buf, sem).wait()
        pltpu.sync_copy(buf, word_out.at[pl.ds(off, CHUNK)])
        return carry

    lax.fori_loop(0, NCHUNK, chunk, 0)


def _gather(sku_idx, cat_idx, price_idx, word_idx, sku_t, c2, p2, w2):
    mesh = plsc.VectorSubcoreMesh(core_axis_name="c", subcore_axis_name="s")
    f = functools.partial(
        pl.kernel,
        mesh=mesh,
        compiler_params=pltpu.CompilerParams(use_tc_tiling_on_sc=False),
        out_type=[
            jax.ShapeDtypeStruct((N, SKU_DIM), jnp.float32),
            jax.ShapeDtypeStruct((N, ITEM_DIM), jnp.float32),
            jax.ShapeDtypeStruct((N, ITEM_DIM), jnp.float32),
            jax.ShapeDtypeStruct((N, ITEM_DIM), jnp.float32),
        ],
        scratch_types=[
            pltpu.VMEM((NCHUNK, CHUNK), jnp.int32),
            pltpu.VMEM((NCHUNK, CHUNK), jnp.int32),
            pltpu.VMEM((NCHUNK, CHUNK), jnp.int32),
            pltpu.VMEM((NCHUNK, CHUNK), jnp.int32),
            pltpu.VMEM((CHUNK, SKU_DIM), jnp.float32),
            pltpu.VMEM((CHUNK, ITEM_DIM), jnp.float32),
            pltpu.SemaphoreType.DMA,
        ],
    )(_gather_body)
    return f(sku_idx, cat_idx, price_idx, word_idx, sku_t, c2, p2, w2)


_RBLK = 2048


def _combine_body(sku_rows, c2r, p2r, w2r,
                  sku_g, sku_b, proj_w, proj_b, proj_g, proj_b2, w_s, out):
    x = _ln(sku_rows[...], sku_g[...], sku_b[...])
    x = jnp.dot(x, proj_w[...], preferred_element_type=jnp.float32) + proj_b[...]
    x = jax.nn.relu(_ln(x, proj_g[...], proj_b2[...]))
    x = jnp.dot(x, w_s[...], preferred_element_type=jnp.float32)
    out[...] = jax.nn.relu(x + c2r[...] + p2r[...] + w2r[...])


def _combine(sku_rows, c2r, p2r, w2r, sku_g, sku_b,
             proj_w, proj_b, proj_g, proj_b2, w_s):
    grid = N // _RBLK
    row = lambda d: pl.BlockSpec((_RBLK, d), lambda i: (i, 0))
    full = lambda shape: pl.BlockSpec(shape, lambda i: (0, 0))
    return pl.pallas_call(
        _combine_body,
        grid=(grid,),
        in_specs=[
            row(SKU_DIM), row(ITEM_DIM), row(ITEM_DIM), row(ITEM_DIM),
            full((1, SKU_DIM)), full((1, SKU_DIM)),
            full((SKU_DIM, HID)), full((1, HID)), full((1, HID)), full((1, HID)),
            full((HID, ITEM_DIM)),
        ],
        out_specs=row(ITEM_DIM),
        out_shape=jax.ShapeDtypeStruct((N, ITEM_DIM), jnp.float32),
    )(sku_rows, c2r, p2r, w2r,
      sku_g.reshape(1, SKU_DIM), sku_b.reshape(1, SKU_DIM),
      proj_w, proj_b.reshape(1, HID), proj_g.reshape(1, HID),
      proj_b2.reshape(1, HID), w_s)


def kernel(sku_id, cat_id, price_id, word_ids, sku_table, sku_ln_g, sku_ln_b,
           proj_W, proj_b, proj_ln_g, proj_ln_b, cat_table, cat_ln_g,
           cat_ln_b, price_table, price_ln_g, price_ln_b, word_table,
           fc1_W, fc1_b):
    c2, p2, w2 = _prep(cat_table, cat_ln_g, cat_ln_b,
                       price_table, price_ln_g, price_ln_b,
                       word_table, fc1_W, fc1_b)
    shape_ids = lambda a: a.reshape(NW, NCHUNK, CHUNK).astype(jnp.int32)
    sku_rows, c2r, p2r, w2r = _gather(
        shape_ids(sku_id), shape_ids(cat_id), shape_ids(price_id),
        shape_ids(word_ids), sku_table, c2, p2, w2)
    out = _combine(sku_rows, c2r, p2r, w2r, sku_ln_g, sku_ln_b,
                   proj_W, proj_b, proj_ln_g, proj_ln_b, fc1_W[:HID, :])
    return out.reshape(B, L, ITEM_DIM)
